# baseline (speedup 1.0000x reference)
import functools

import jax
import jax.numpy as jnp
from jax import lax
from jax.experimental import pallas as pl
from jax.experimental.pallas import tpu as pltpu
from jax.experimental.pallas import tpu_sc as plsc

_N = 16384
_K = 8192
_D = 256
_TN = 256
_BETA = 0.25
_SEG = (0, 2736, 5472, 8192)


def _bf16r(x):
    return x.astype(jnp.bfloat16).astype(jnp.float32)


def _vq_body(n_tiles, n_tokens, k, d, tn,
             z_ref, zz_ref, cb_ref, cc_ref,
             idx_ref, enc_ref, loss_ref, perp_ref,
             colsum_ref, lcol_ref):
    i = pl.program_id(0)
    z = z_ref[...]
    cb = cb_ref[...]
    e = lax.dot_general(z, cb, dimension_numbers=(((1,), (1,)), ((), ())),
                        preferred_element_type=jnp.float32)
    dis = (zz_ref[...] - 2.0 * e) + cc_ref[...]
    ii = lax.broadcasted_iota(jnp.int32, (tn, k), 1)
    iif = ii.astype(jnp.float32)
    ge2 = ii >= _SEG[1]
    ge3 = ii >= _SEG[2]
    inf = jnp.float32(jnp.inf)
    m1 = jnp.min(jnp.where(ge2, inf, dis), axis=1, keepdims=True)
    m2 = jnp.min(jnp.where(ge2 ^ ge3, dis, inf), axis=1, keepdims=True)
    m3 = jnp.min(jnp.where(ge3, dis, inf), axis=1, keepdims=True)
    row_min = jnp.minimum(m1, jnp.minimum(m2, m3))
    acc_v = _bf16r(m1)
    t2 = m2 < acc_v
    acc_v = _bf16r(jnp.where(t2, m2, acc_v))
    t3 = m3 < acc_v
    m_win = jnp.where(t3, m3, jnp.where(t2, m2, m1))
    lo = jnp.where(t3, _SEG[2], jnp.where(t2, _SEG[1], _SEG[0]))
    hi = jnp.where(t3, _SEG[3], jnp.where(t2, _SEG[2], _SEG[1]))
    in_win = (ii - lo).astype(jnp.uint32) < (hi - lo).astype(jnp.uint32)
    cand = jnp.where(in_win & (dis == m_win), iif, jnp.float32(2.0 * k))
    amin = jnp.min(cand, axis=1, keepdims=True).astype(jnp.int32)
    idx_ref[...] = amin
    enc = (ii == amin).astype(jnp.int32)
    enc_ref[...] = enc
    colsum = jnp.sum(enc, axis=0, keepdims=True)

    @pl.when(i == 0)
    def _():
        colsum_ref[...] = colsum
        lcol_ref[...] = row_min

    @pl.when(i > 0)
    def _():
        colsum_ref[...] += colsum
        lcol_ref[...] += row_min

    @pl.when(i == n_tiles - 1)
    def _():
        e_mean = colsum_ref[...].astype(jnp.float32) / n_tokens
        ent = jnp.sum(e_mean * jnp.log(e_mean + 1e-10))
        perp_ref[0, 0] = jnp.exp(-ent)
        loss_ref[0, 0] = _BETA * jnp.sum(lcol_ref[...]) / (n_tokens * d)


def _make_distance_kernel(n_tokens, k, d, tn, interpret=False):
    n_tiles = n_tokens // tn
    body = functools.partial(_vq_body, n_tiles, n_tokens, k, d, tn)
    return pl.pallas_call(
        body,
        grid=(n_tiles,),
        in_specs=[
            pl.BlockSpec((tn, d), lambda i: (i, 0)),
            pl.BlockSpec((tn, 1), lambda i: (i, 0)),
            pl.BlockSpec((k, d), lambda i: (0, 0)),
            pl.BlockSpec((1, k), lambda i: (0, 0)),
        ],
        out_specs=[
            pl.BlockSpec((tn, 1), lambda i: (i, 0)),
            pl.BlockSpec((tn, k), lambda i: (i, 0)),
            pl.BlockSpec(memory_space=pltpu.SMEM),
            pl.BlockSpec(memory_space=pltpu.SMEM),
        ],
        out_shape=[
            jax.ShapeDtypeStruct((n_tokens, 1), jnp.int32),
            jax.ShapeDtypeStruct((n_tokens, k), jnp.int32),
            jax.ShapeDtypeStruct((1, 1), jnp.float32),
            jax.ShapeDtypeStruct((1, 1), jnp.float32),
        ],
        scratch_shapes=[
            pltpu.VMEM((1, k), jnp.int32),
            pltpu.VMEM((tn, 1), jnp.float32),
        ],
        compiler_params=pltpu.CompilerParams(
            dimension_semantics=("arbitrary",)),
        interpret=interpret,
    )


def _make_sc_gather(n_rows, d):
    info = plsc.get_sparse_core_info()
    nw = info.num_cores * info.num_subcores
    rows_per_w = n_rows // nw
    chunk = 256
    n_chunks = rows_per_w // chunk
    mesh = plsc.VectorSubcoreMesh(core_axis_name="c", subcore_axis_name="s")

    @functools.partial(
        pl.kernel, mesh=mesh,
        out_type=jax.ShapeDtypeStruct((n_rows, d), jnp.float32),
        scratch_types=[
            pltpu.VMEM((chunk,), jnp.int32),
            pltpu.VMEM((chunk, d), jnp.float32),
            pltpu.SemaphoreType.DMA,
        ],
    )
    def gather(cb_hbm, idx_hbm, out_hbm, idx_v, rows_v, sem):
        wid = lax.axis_index("s") * info.num_cores + lax.axis_index("c")
        for c in range(n_chunks):
            base = wid * rows_per_w + c * chunk
            pltpu.sync_copy(idx_hbm.at[pl.ds(base, chunk)], idx_v)
            pltpu.async_copy(cb_hbm.at[idx_v], rows_v, sem).wait()
            pltpu.sync_copy(rows_v, out_hbm.at[pl.ds(base, chunk)])

    return gather


def kernel(z_e, codebook):
    b, c, h, w = z_e.shape
    z_flat = jnp.transpose(z_e, (0, 2, 3, 1)).reshape(-1, _D)
    zz = jnp.sum(z_flat ** 2, axis=1, keepdims=True)
    cc = jnp.sum(codebook ** 2, axis=1, keepdims=True).T
    idx2, enc, loss, perp = _make_distance_kernel(_N, _K, _D, _TN)(
        z_flat, zz, codebook, cc)
    z_q = _make_sc_gather(_N, _D)(codebook, idx2.reshape(-1))
    z_q_out = jnp.transpose(z_q.reshape(b, h, w, c), (0, 3, 1, 2))
    return (loss[0, 0], z_q_out, perp[0, 0], enc, idx2)


# speedup vs baseline: 1.0278x; 1.0278x over previous
import functools

import jax
import jax.numpy as jnp
from jax import lax
from jax.experimental import pallas as pl
from jax.experimental.pallas import tpu as pltpu
from jax.experimental.pallas import tpu_sc as plsc

_N = 16384
_K = 8192
_D = 256
_TN = 256
_BETA = 0.25
_SEG = (0, 2736, 5472, 8192)


def _bf16r(x):
    return x.astype(jnp.bfloat16).astype(jnp.float32)


def _vq_body(n_tiles, n_tokens, k, d, tn,
             z_ref, zz_ref, cb_ref, cc_ref,
             idx_ref, enc_ref, loss_ref, perp_ref,
             colsum_ref, lcol_ref):
    i = pl.program_id(0)
    z = z_ref[...]
    cb = cb_ref[...]
    e = lax.dot_general(z, cb, dimension_numbers=(((1,), (1,)), ((), ())),
                        preferred_element_type=jnp.float32)
    dis = (zz_ref[...] - 2.0 * e) + cc_ref[...]
    ii = lax.broadcasted_iota(jnp.int32, (tn, k), 1)
    iif = ii.astype(jnp.float32)
    ge2 = ii >= _SEG[1]
    ge3 = ii >= _SEG[2]
    inf = jnp.float32(jnp.inf)
    m1 = jnp.min(jnp.where(ge2, inf, dis), axis=1, keepdims=True)
    m2 = jnp.min(jnp.where(ge2 ^ ge3, dis, inf), axis=1, keepdims=True)
    m3 = jnp.min(jnp.where(ge3, dis, inf), axis=1, keepdims=True)
    row_min = jnp.minimum(m1, jnp.minimum(m2, m3))
    acc_v = _bf16r(m1)
    t2 = m2 < acc_v
    acc_v = _bf16r(jnp.where(t2, m2, acc_v))
    t3 = m3 < acc_v
    m_win = jnp.where(t3, m3, jnp.where(t2, m2, m1))
    lo = jnp.where(t3, _SEG[2], jnp.where(t2, _SEG[1], _SEG[0]))
    hi = jnp.where(t3, _SEG[3], jnp.where(t2, _SEG[2], _SEG[1]))
    in_win = (ii - lo).astype(jnp.uint32) < (hi - lo).astype(jnp.uint32)
    cand = jnp.where(in_win & (dis == m_win), iif, jnp.float32(2.0 * k))
    amin = jnp.min(cand, axis=1, keepdims=True).astype(jnp.int32)
    idx_ref[...] = amin
    encm = ii == amin
    enc_ref[...] = encm.astype(jnp.int32)
    encf = encm.astype(jnp.float32)
    ones = jnp.ones((1, tn), jnp.float32)
    colsum = lax.dot_general(ones, encf,
                             dimension_numbers=(((1,), (0,)), ((), ())),
                             preferred_element_type=jnp.float32)

    @pl.when(i == 0)
    def _():
        colsum_ref[...] = colsum
        lcol_ref[...] = row_min

    @pl.when(i > 0)
    def _():
        colsum_ref[...] += colsum
        lcol_ref[...] += row_min

    @pl.when(i == n_tiles - 1)
    def _():
        e_mean = colsum_ref[...] / n_tokens
        ent = jnp.sum(e_mean * jnp.log(e_mean + 1e-10))
        perp_ref[0, 0] = jnp.exp(-ent)
        loss_ref[0, 0] = _BETA * jnp.sum(lcol_ref[...]) / (n_tokens * d)


def _make_distance_kernel(n_tokens, k, d, tn, interpret=False):
    n_tiles = n_tokens // tn
    body = functools.partial(_vq_body, n_tiles, n_tokens, k, d, tn)
    return pl.pallas_call(
        body,
        grid=(n_tiles,),
        in_specs=[
            pl.BlockSpec((tn, d), lambda i: (i, 0)),
            pl.BlockSpec((tn, 1), lambda i: (i, 0)),
            pl.BlockSpec((k, d), lambda i: (0, 0)),
            pl.BlockSpec((1, k), lambda i: (0, 0)),
        ],
        out_specs=[
            pl.BlockSpec((tn, 1), lambda i: (i, 0)),
            pl.BlockSpec((tn, k), lambda i: (i, 0)),
            pl.BlockSpec(memory_space=pltpu.SMEM),
            pl.BlockSpec(memory_space=pltpu.SMEM),
        ],
        out_shape=[
            jax.ShapeDtypeStruct((n_tokens, 1), jnp.int32),
            jax.ShapeDtypeStruct((n_tokens, k), jnp.int32),
            jax.ShapeDtypeStruct((1, 1), jnp.float32),
            jax.ShapeDtypeStruct((1, 1), jnp.float32),
        ],
        scratch_shapes=[
            pltpu.VMEM((1, k), jnp.float32),
            pltpu.VMEM((tn, 1), jnp.float32),
        ],
        compiler_params=pltpu.CompilerParams(
            dimension_semantics=("arbitrary",)),
        interpret=interpret,
    )


def _make_sc_gather(n_rows, d):
    info = plsc.get_sparse_core_info()
    nw = info.num_cores * info.num_subcores
    rows_per_w = n_rows // nw
    chunk = 256
    n_chunks = rows_per_w // chunk
    mesh = plsc.VectorSubcoreMesh(core_axis_name="c", subcore_axis_name="s")

    @functools.partial(
        pl.kernel, mesh=mesh,
        out_type=jax.ShapeDtypeStruct((n_rows, d), jnp.float32),
        scratch_types=[
            pltpu.VMEM((chunk,), jnp.int32),
            pltpu.VMEM((chunk, d), jnp.float32),
            pltpu.SemaphoreType.DMA,
        ],
    )
    def gather(cb_hbm, idx_hbm, out_hbm, idx_v, rows_v, sem):
        wid = lax.axis_index("s") * info.num_cores + lax.axis_index("c")
        for c in range(n_chunks):
            base = wid * rows_per_w + c * chunk
            pltpu.sync_copy(idx_hbm.at[pl.ds(base, chunk)], idx_v)
            pltpu.async_copy(cb_hbm.at[idx_v], rows_v, sem).wait()
            pltpu.sync_copy(rows_v, out_hbm.at[pl.ds(base, chunk)])

    return gather


def kernel(z_e, codebook):
    b, c, h, w = z_e.shape
    z_flat = jnp.transpose(z_e, (0, 2, 3, 1)).reshape(-1, _D)
    zz = jnp.sum(z_flat ** 2, axis=1, keepdims=True)
    cc = jnp.sum(codebook ** 2, axis=1, keepdims=True).T
    idx2, enc, loss, perp = _make_distance_kernel(_N, _K, _D, _TN)(
        z_flat, zz, codebook, cc)
    z_q = _make_sc_gather(_N, _D)(codebook, idx2.reshape(-1))
    z_q_out = jnp.transpose(z_q.reshape(b, h, w, c), (0, 3, 1, 2))
    return (loss[0, 0], z_q_out, perp[0, 0], enc, idx2)
